# baseline (speedup 1.0000x reference)
import functools

import jax
import jax.numpy as jnp
from jax import lax
from jax.experimental import pallas as pl
from jax.experimental.pallas import tpu as pltpu
from jax.experimental.pallas import tpu_sc as plsc

N = 10000
E = 320000
D = 128
Z = 16

NC = 2
NS = 16
NW = NC * NS
EPW = E // NW
CHUNK = 40
NCHUNK = EPW // CHUNK
NB = 5
CHUNKW = 125
NCHUNKW = EPW // CHUNKW
NPAD = 10240
RPT = NPAD // NS
ZROWS = 128

ROW_BLK = 1000
GRID = N // ROW_BLK


def _make_edge_pass(feat, chunk, nchunk):
  mesh = plsc.VectorSubcoreMesh(core_axis_name="c", subcore_axis_name="s")

  @functools.partial(
      pl.kernel,
      mesh=mesh,
      out_type=jax.ShapeDtypeStruct((NC * NPAD, feat), jnp.float32),
      compiler_params=pltpu.CompilerParams(use_tc_tiling_on_sc=False),
      scratch_types=[
          pltpu.VMEM((nchunk, chunk), jnp.int32),
          pltpu.VMEM((nchunk, chunk), jnp.int32),
          pltpu.VMEM((NB, chunk, feat), jnp.float32),
          pltpu.VMEM_SHARED((NPAD, feat), jnp.float32),
      ] + [pltpu.SemaphoreType.DMA] * NB,
  )
  def k(src_hbm, dst_hbm, ht_hbm, out_hbm, sbuf, dbuf, rows, acc, *gsems):
    c = lax.axis_index("c")
    s = lax.axis_index("s")
    wid = s * NC + c

    pltpu.sync_copy(src_hbm.at[wid], sbuf)
    pltpu.sync_copy(dst_hbm.at[wid], dbuf)

    def zrow(i, carry):
      for q in range(feat // 16):
        rows[0, i, pl.ds(q * 16, 16)] = jnp.zeros((16,), jnp.float32)
      return carry

    lax.fori_loop(0, chunk, zrow, 0)
    for t in range(RPT // chunk if RPT % chunk == 0 else 0):
      pltpu.sync_copy(rows.at[0], acc.at[pl.ds(s * RPT + t * chunk, chunk)])
    if RPT % chunk:
      nz = RPT // 16
      def zcopy(t, carry):
        pltpu.sync_copy(rows.at[0, pl.ds(0, 16)],
                        acc.at[pl.ds(s * RPT + t * 16, 16)])
        return carry
      lax.fori_loop(0, nz, zcopy, 0)
    plsc.subcore_barrier()

    for b in range(NB):
      pltpu.async_copy(ht_hbm.at[sbuf.at[b]], rows.at[b], gsems[b])

    def outer(g, carry):
      jb = g * NB
      for b in range(NB):
        j = jb + b
        pltpu.make_async_copy(ht_hbm.at[sbuf.at[j]], rows.at[b],
                              gsems[b]).wait()
        pltpu.sync_copy(rows.at[b], acc.at[dbuf.at[j]], add=True)

        @pl.when(j + NB < nchunk)
        def _():
          pltpu.async_copy(ht_hbm.at[sbuf.at[j + NB]], rows.at[b], gsems[b])

      return carry

    lax.fori_loop(0, nchunk // NB, outer, 0)
    plsc.subcore_barrier()
    pltpu.sync_copy(acc.at[pl.ds(s * RPT, RPT)],
                    out_hbm.at[pl.ds(c * NPAD + s * RPT, RPT)])

  return k


def _make_deg_pass():
  mesh = plsc.VectorSubcoreMesh(core_axis_name="c", subcore_axis_name="s")

  @functools.partial(
      pl.kernel,
      mesh=mesh,
      out_type=jax.ShapeDtypeStruct((NC * NPAD, Z), jnp.float32),
      compiler_params=pltpu.CompilerParams(use_tc_tiling_on_sc=False),
      scratch_types=[
          pltpu.VMEM((NCHUNKW, CHUNKW), jnp.int32),
          pltpu.VMEM((CHUNKW, Z), jnp.float32),
          pltpu.VMEM((ZROWS, Z), jnp.float32),
          pltpu.VMEM_SHARED((NPAD, Z), jnp.float32),
          pltpu.SemaphoreType.DMA,
      ],
  )
  def k(dst_hbm, out_hbm, dbuf, ones, zbuf, acc, ssem):
    c = lax.axis_index("c")
    s = lax.axis_index("s")
    wid = s * NC + c

    pltpu.sync_copy(dst_hbm.at[wid], dbuf)

    def fill(i, carry):
      zbuf[i, pl.ds(0, 16)] = jnp.zeros((16,), jnp.float32)
      return carry

    lax.fori_loop(0, ZROWS, fill, 0)

    def fill1(i, carry):
      ones[i, pl.ds(0, 16)] = jnp.ones((16,), jnp.float32)
      return carry

    lax.fori_loop(0, CHUNKW, fill1, 0)
    for t in range(RPT // ZROWS):
      pltpu.sync_copy(zbuf, acc.at[pl.ds(s * RPT + t * ZROWS, ZROWS)])
    plsc.subcore_barrier()

    def outer(g, carry):
      jb = g * NB
      for b in range(NB):
        pltpu.async_copy(ones, acc.at[dbuf.at[jb + b]], ssem, add=True)
      for b in range(NB):
        pltpu.make_async_copy(ones, acc.at[dbuf.at[jb + b]], ssem).wait()
      return carry

    lax.fori_loop(0, NCHUNKW // NB, outer, 0)
    plsc.subcore_barrier()
    pltpu.sync_copy(acc.at[pl.ds(s * RPT, RPT)],
                    out_hbm.at[pl.ds(c * NPAD + s * RPT, RPT)])

  return k


def _make_edge_pass_spmem(feat, chunk, nchunk):
  mesh = plsc.VectorSubcoreMesh(core_axis_name="c", subcore_axis_name="s")
  nrows = N // NS

  @functools.partial(
      pl.kernel,
      mesh=mesh,
      out_type=jax.ShapeDtypeStruct((NC * NPAD, feat), jnp.float32),
      compiler_params=pltpu.CompilerParams(use_tc_tiling_on_sc=False),
      scratch_types=[
          pltpu.VMEM((nchunk, chunk), jnp.int32),
          pltpu.VMEM((nchunk, chunk), jnp.int32),
          pltpu.VMEM((NB, chunk, feat), jnp.float32),
          pltpu.VMEM_SHARED((N, feat), jnp.float32),
          pltpu.VMEM_SHARED((NPAD, feat), jnp.float32),
      ] + [pltpu.SemaphoreType.DMA] * NB,
  )
  def k(src_hbm, dst_hbm, ht_hbm, out_hbm, sbuf, dbuf, rows, tab, acc,
        *gsems):
    c = lax.axis_index("c")
    s = lax.axis_index("s")
    wid = s * NC + c

    pltpu.sync_copy(src_hbm.at[wid], sbuf)
    pltpu.sync_copy(dst_hbm.at[wid], dbuf)
    pltpu.sync_copy(ht_hbm.at[pl.ds(s * nrows, nrows)],
                    tab.at[pl.ds(s * nrows, nrows)])

    def zrow(i, carry):
      for q in range(feat // 16):
        rows[0, i, pl.ds(q * 16, 16)] = jnp.zeros((16,), jnp.float32)
      return carry

    lax.fori_loop(0, chunk, zrow, 0)
    for t in range(RPT // chunk if RPT % chunk == 0 else 0):
      pltpu.sync_copy(rows.at[0], acc.at[pl.ds(s * RPT + t * chunk, chunk)])
    if RPT % chunk:
      nz = RPT // 16

      def zcopy(t, carry):
        pltpu.sync_copy(rows.at[0, pl.ds(0, 16)],
                        acc.at[pl.ds(s * RPT + t * 16, 16)])
        return carry

      lax.fori_loop(0, nz, zcopy, 0)
    plsc.subcore_barrier()

    for b in range(NB):
      pltpu.async_copy(tab.at[sbuf.at[b]], rows.at[b], gsems[b])

    def outer(g, carry):
      jb = g * NB
      for b in range(NB):
        j = jb + b
        pltpu.make_async_copy(tab.at[sbuf.at[j]], rows.at[b],
                              gsems[b]).wait()
        pltpu.sync_copy(rows.at[b], acc.at[dbuf.at[j]], add=True)

        @pl.when(j + NB < nchunk)
        def _():
          pltpu.async_copy(tab.at[sbuf.at[j + NB]], rows.at[b], gsems[b])

      return carry

    lax.fori_loop(0, nchunk // NB, outer, 0)
    plsc.subcore_barrier()
    pltpu.sync_copy(acc.at[pl.ds(s * RPT, RPT)],
                    out_hbm.at[pl.ds(c * NPAD + s * RPT, RPT)])

  return k


DH = D // NC
EPT = E // NS
CH2 = 125
NCH2 = EPT // CH2
IDXBLK = 20
NBLK = NCH2 // IDXBLK
NB2 = 4
TROWS = N // NS


def _make_edge_pass_split():
  mesh = plsc.VectorSubcoreMesh(core_axis_name="c", subcore_axis_name="s")

  @functools.partial(
      pl.kernel,
      mesh=mesh,
      out_type=jax.ShapeDtypeStruct((NC * NPAD, DH), jnp.float32),
      compiler_params=pltpu.CompilerParams(use_tc_tiling_on_sc=False),
      scratch_types=[
          pltpu.VMEM((2, IDXBLK, CH2), jnp.int32),
          pltpu.VMEM((2, IDXBLK, CH2), jnp.int32),
          pltpu.VMEM((NB2, CH2, DH), jnp.float32),
          pltpu.VMEM_SHARED((N, DH), jnp.float32),
          pltpu.VMEM_SHARED((NPAD, DH), jnp.float32),
      ] + [pltpu.SemaphoreType.DMA] * NB2,
  )
  def k(src_hbm, dst_hbm, ht_hbm, out_hbm, sblk, dblk, rows, tab, acc,
        *gsems):
    c = lax.axis_index("c")
    s = lax.axis_index("s")

    pltpu.sync_copy(ht_hbm.at[c, pl.ds(s * TROWS, TROWS)],
                    tab.at[pl.ds(s * TROWS, TROWS)])

    def zrow(i, carry):
      for q in range(DH // 16):
        rows[0, i, pl.ds(q * 16, 16)] = jnp.zeros((16,), jnp.float32)
      return carry

    lax.fori_loop(0, CH2, zrow, 0)

    def zcopy(t, carry):
      pltpu.sync_copy(rows.at[0, pl.ds(0, 16)],
                      acc.at[pl.ds(s * RPT + t * 16, 16)])
      return carry

    lax.fori_loop(0, RPT // 16, zcopy, 0)
    plsc.subcore_barrier()

    for blk in range(NBLK):
      pb = blk % 2
      pltpu.sync_copy(src_hbm.at[s, pl.ds(blk * IDXBLK, IDXBLK)],
                      sblk.at[pb])
      pltpu.sync_copy(dst_hbm.at[s, pl.ds(blk * IDXBLK, IDXBLK)],
                      dblk.at[pb])
      for b in range(NB2):
        pltpu.async_copy(tab.at[sblk.at[pb, b]], rows.at[b], gsems[b])

      def grp(g, carry):
        jb = g * NB2
        for b in range(NB2):
          j = jb + b
          pltpu.make_async_copy(tab.at[sblk.at[pb, j]], rows.at[b],
                                gsems[b]).wait()
          pltpu.sync_copy(rows.at[b], acc.at[dblk.at[pb, j]], add=True)

          @pl.when(j + NB2 < IDXBLK)
          def _():
            pltpu.async_copy(tab.at[sblk.at[pb, j + NB2]], rows.at[b],
                             gsems[b])

        return carry

      lax.fori_loop(0, IDXBLK // NB2 - 1, grp, 0)
      for b in range(NB2):
        j = IDXBLK - NB2 + b
        pltpu.make_async_copy(tab.at[sblk.at[pb, j]], rows.at[b],
                              gsems[b]).wait()
        pltpu.sync_copy(rows.at[b], acc.at[dblk.at[pb, j]], add=True)

    plsc.subcore_barrier()
    pltpu.sync_copy(acc.at[pl.ds(s * RPT, RPT)],
                    out_hbm.at[pl.ds(c * NPAD + s * RPT, RPT)])

  return k


_edge_pass_d = _make_edge_pass_split()
_edge_pass_z = _make_edge_pass_spmem(Z, CHUNKW, NCHUNKW)
_deg_pass = _make_deg_pass()


def _dinv_from(dega, degb):
  deg = dega[0, :, 0] + degb[0, :, 0] + 1.0
  return lax.rsqrt(jnp.maximum(deg, 1e-12))


def _mlp_body(x, w1, b1, w2, b2, w0, dega, degb, out):
  dinv = _dinv_from(dega[...], degb[...])
  h = jax.nn.sigmoid(jnp.dot(x[...], w1[...],
                             preferred_element_type=jnp.float32) + b1[...])
  h = jax.nn.sigmoid(jnp.dot(h, w2[...],
                             preferred_element_type=jnp.float32) + b2[...])
  t = dinv[:, None] * jnp.dot(h, w0[...],
                              preferred_element_type=jnp.float32)
  out[0] = t[:, :DH]
  out[1] = t[:, DH:]


def _combine128_body(sp, hts, b, w, dega, degb, out):
  dinv = _dinv_from(dega[...], degb[...])
  full = jnp.concatenate([sp[0] + hts[0], sp[1] + hts[1]], axis=1)
  o = dinv[:, None] * full + b[...]
  t = dinv[:, None] * jnp.dot(o, w[...],
                              preferred_element_type=jnp.float32)
  out[0] = t[:, :DH]
  out[1] = t[:, DH:]


def _combinez_body(sp, hts, b, w, dega, degb, out):
  dinv = _dinv_from(dega[...], degb[...])
  full = jnp.concatenate([sp[0] + hts[0], sp[1] + hts[1]], axis=1)
  o = dinv[:, None] * full + b[...]
  out[...] = dinv[:, None] * jnp.dot(o, w[...],
                                     preferred_element_type=jnp.float32)


def _final_body(spa, spb, ht, b, dega, degb, out):
  dinv = _dinv_from(dega[...], degb[...])
  out[...] = dinv[:, None] * (spa[0] + spb[0] + ht[...]) + b[...]


def _row_spec(feat):
  return pl.BlockSpec((ROW_BLK, feat), lambda i: (i, 0))


def _split_spec(feat):
  return pl.BlockSpec((NC, ROW_BLK, feat), lambda i: (0, i, 0))


def _slab_specs(feat):
  return (pl.BlockSpec((1, ROW_BLK, feat), lambda i: (0, i, 0)),
          pl.BlockSpec((1, ROW_BLK, feat), lambda i: (1, i, 0)))


def _full_spec(shape):
  return pl.BlockSpec(shape, lambda i: tuple(0 for _ in shape))


def _tc_mlp(X, w1, b1, w2, b2, w0, degp):
  dega, degb = _slab_specs(Z)
  return pl.pallas_call(
      _mlp_body,
      grid=(GRID,),
      in_specs=[
          _row_spec(D), _full_spec((D, D)), _full_spec((D,)),
          _full_spec((D, D)), _full_spec((D,)), _full_spec((D, D)),
          dega, degb,
      ],
      out_specs=_split_spec(DH),
      out_shape=jax.ShapeDtypeStruct((NC, N, DH), jnp.float32),
  )(X, w1, b1, w2, b2, w0, degp, degp)


def _tc_combine128(sp, hts, b, w, degp):
  dega, degb = _slab_specs(Z)
  return pl.pallas_call(
      _combine128_body,
      grid=(GRID,),
      in_specs=[
          _split_spec(DH), _split_spec(DH), _full_spec((D,)),
          _full_spec((D, D)), dega, degb,
      ],
      out_specs=_split_spec(DH),
      out_shape=jax.ShapeDtypeStruct((NC, N, DH), jnp.float32),
  )(sp, hts, b, w, degp, degp)


def _tc_combinez(sp, hts, b, w, degp):
  dega, degb = _slab_specs(Z)
  return pl.pallas_call(
      _combinez_body,
      grid=(GRID,),
      in_specs=[
          _split_spec(DH), _split_spec(DH), _full_spec((D,)),
          _full_spec((D, Z)), dega, degb,
      ],
      out_specs=_row_spec(Z),
      out_shape=jax.ShapeDtypeStruct((N, Z), jnp.float32),
  )(sp, hts, b, w, degp, degp)


def _tc_final(sp, ht, b, degp):
  spa, spb = _slab_specs(Z)
  dega, degb = _slab_specs(Z)
  return pl.pallas_call(
      _final_body,
      grid=(GRID,),
      in_specs=[spa, spb, _row_spec(Z), _full_spec((Z,)), dega, degb],
      out_specs=_row_spec(Z),
      out_shape=jax.ShapeDtypeStruct((N, Z), jnp.float32),
  )(sp, sp, ht, b, degp, degp)


def kernel(adj, X, fc1_W, fc1_b, fc2_W, fc2_b, gcn0_W, gcn0_b, gcn1_W,
           gcn1_b, assign_W, assign_b):
  src = adj[0].astype(jnp.int32)
  dst = adj[1].astype(jnp.int32)
  src_t = src.reshape(NS, NCH2, CH2)
  dst_t = dst.reshape(NS, NCH2, CH2)
  src_w = src.reshape(NW, NCHUNKW, CHUNKW)
  dst_w = dst.reshape(NW, NCHUNKW, CHUNKW)

  degp = _deg_pass(dst_w).reshape(NC, NPAD, Z)
  hts0 = _tc_mlp(X, fc1_W, fc1_b, fc2_W, fc2_b, gcn0_W, degp)
  sp0 = _edge_pass_d(src_t, dst_t, hts0).reshape(NC, NPAD, DH)
  hts1 = _tc_combine128(sp0, hts0, gcn0_b, gcn1_W, degp)
  sp1 = _edge_pass_d(src_t, dst_t, hts1).reshape(NC, NPAD, DH)
  ht2 = _tc_combinez(sp1, hts1, gcn1_b, assign_W, degp)
  sp2 = _edge_pass_z(src_w, dst_w, ht2).reshape(NC, NPAD, Z)
  return _tc_final(sp2, ht2, assign_b, degp)


# speedup vs baseline: 1.5982x; 1.5982x over previous
import functools

import jax
import jax.numpy as jnp
from jax import lax
from jax.experimental import pallas as pl
from jax.experimental.pallas import tpu as pltpu
from jax.experimental.pallas import tpu_sc as plsc

N = 10000
E = 320000
D = 128
Z = 16

NC = 2
NS = 16
NW = NC * NS
EPW = E // NW
CHUNK = 40
NCHUNK = EPW // CHUNK
NB = 5
CHUNKW = 125
NCHUNKW = EPW // CHUNKW
NPAD = 10240
RPT = NPAD // NS
ZROWS = 128

ROW_BLK = 1000
GRID = N // ROW_BLK


def _make_edge_pass(feat, chunk, nchunk):
  mesh = plsc.VectorSubcoreMesh(core_axis_name="c", subcore_axis_name="s")

  @functools.partial(
      pl.kernel,
      mesh=mesh,
      out_type=jax.ShapeDtypeStruct((NC * NPAD, feat), jnp.float32),
      compiler_params=pltpu.CompilerParams(use_tc_tiling_on_sc=False),
      scratch_types=[
          pltpu.VMEM((nchunk, chunk), jnp.int32),
          pltpu.VMEM((nchunk, chunk), jnp.int32),
          pltpu.VMEM((NB, chunk, feat), jnp.float32),
          pltpu.VMEM_SHARED((NPAD, feat), jnp.float32),
      ] + [pltpu.SemaphoreType.DMA] * NB,
  )
  def k(src_hbm, dst_hbm, ht_hbm, out_hbm, sbuf, dbuf, rows, acc, *gsems):
    c = lax.axis_index("c")
    s = lax.axis_index("s")
    wid = s * NC + c

    pltpu.sync_copy(src_hbm.at[wid], sbuf)
    pltpu.sync_copy(dst_hbm.at[wid], dbuf)

    def zrow(i, carry):
      for q in range(feat // 16):
        rows[0, i, pl.ds(q * 16, 16)] = jnp.zeros((16,), jnp.float32)
      return carry

    lax.fori_loop(0, chunk, zrow, 0)
    for t in range(RPT // chunk if RPT % chunk == 0 else 0):
      pltpu.sync_copy(rows.at[0], acc.at[pl.ds(s * RPT + t * chunk, chunk)])
    if RPT % chunk:
      nz = RPT // 16
      def zcopy(t, carry):
        pltpu.sync_copy(rows.at[0, pl.ds(0, 16)],
                        acc.at[pl.ds(s * RPT + t * 16, 16)])
        return carry
      lax.fori_loop(0, nz, zcopy, 0)
    plsc.subcore_barrier()

    for b in range(NB):
      pltpu.async_copy(ht_hbm.at[sbuf.at[b]], rows.at[b], gsems[b])

    def outer(g, carry):
      jb = g * NB
      for b in range(NB):
        j = jb + b
        pltpu.make_async_copy(ht_hbm.at[sbuf.at[j]], rows.at[b],
                              gsems[b]).wait()
        pltpu.sync_copy(rows.at[b], acc.at[dbuf.at[j]], add=True)

        @pl.when(j + NB < nchunk)
        def _():
          pltpu.async_copy(ht_hbm.at[sbuf.at[j + NB]], rows.at[b], gsems[b])

      return carry

    lax.fori_loop(0, nchunk // NB, outer, 0)
    plsc.subcore_barrier()
    pltpu.sync_copy(acc.at[pl.ds(s * RPT, RPT)],
                    out_hbm.at[pl.ds(c * NPAD + s * RPT, RPT)])

  return k


def _make_deg_pass():
  mesh = plsc.VectorSubcoreMesh(core_axis_name="c", subcore_axis_name="s")

  @functools.partial(
      pl.kernel,
      mesh=mesh,
      out_type=jax.ShapeDtypeStruct((NC * NPAD, Z), jnp.float32),
      compiler_params=pltpu.CompilerParams(use_tc_tiling_on_sc=False),
      scratch_types=[
          pltpu.VMEM((NCHUNKW, CHUNKW), jnp.int32),
          pltpu.VMEM((CHUNKW, Z), jnp.float32),
          pltpu.VMEM((ZROWS, Z), jnp.float32),
          pltpu.VMEM_SHARED((NPAD, Z), jnp.float32),
          pltpu.SemaphoreType.DMA,
      ],
  )
  def k(dst_hbm, out_hbm, dbuf, ones, zbuf, acc, ssem):
    c = lax.axis_index("c")
    s = lax.axis_index("s")
    wid = s * NC + c

    pltpu.sync_copy(dst_hbm.at[wid], dbuf)

    def fill(i, carry):
      zbuf[i, pl.ds(0, 16)] = jnp.zeros((16,), jnp.float32)
      return carry

    lax.fori_loop(0, ZROWS, fill, 0)

    def fill1(i, carry):
      ones[i, pl.ds(0, 16)] = jnp.ones((16,), jnp.float32)
      return carry

    lax.fori_loop(0, CHUNKW, fill1, 0)
    for t in range(RPT // ZROWS):
      pltpu.sync_copy(zbuf, acc.at[pl.ds(s * RPT + t * ZROWS, ZROWS)])
    plsc.subcore_barrier()

    def outer(g, carry):
      jb = g * NB
      for b in range(NB):
        pltpu.async_copy(ones, acc.at[dbuf.at[jb + b]], ssem, add=True)
      for b in range(NB):
        pltpu.make_async_copy(ones, acc.at[dbuf.at[jb + b]], ssem).wait()
      return carry

    lax.fori_loop(0, NCHUNKW // NB, outer, 0)
    plsc.subcore_barrier()
    pltpu.sync_copy(acc.at[pl.ds(s * RPT, RPT)],
                    out_hbm.at[pl.ds(c * NPAD + s * RPT, RPT)])

  return k


def _make_edge_pass_spmem(feat, chunk, nchunk):
  mesh = plsc.VectorSubcoreMesh(core_axis_name="c", subcore_axis_name="s")
  nrows = N // NS

  @functools.partial(
      pl.kernel,
      mesh=mesh,
      out_type=jax.ShapeDtypeStruct((NC * NPAD, feat), jnp.float32),
      compiler_params=pltpu.CompilerParams(use_tc_tiling_on_sc=False),
      scratch_types=[
          pltpu.VMEM((nchunk, chunk), jnp.int32),
          pltpu.VMEM((nchunk, chunk), jnp.int32),
          pltpu.VMEM((NB, chunk, feat), jnp.float32),
          pltpu.VMEM_SHARED((N, feat), jnp.float32),
          pltpu.VMEM_SHARED((NPAD, feat), jnp.float32),
      ] + [pltpu.SemaphoreType.DMA] * NB,
  )
  def k(src_hbm, dst_hbm, ht_hbm, out_hbm, sbuf, dbuf, rows, tab, acc,
        *gsems):
    c = lax.axis_index("c")
    s = lax.axis_index("s")
    wid = s * NC + c

    pltpu.sync_copy(src_hbm.at[wid], sbuf)
    pltpu.sync_copy(dst_hbm.at[wid], dbuf)
    pltpu.sync_copy(ht_hbm.at[pl.ds(s * nrows, nrows)],
                    tab.at[pl.ds(s * nrows, nrows)])

    def zrow(i, carry):
      for q in range(feat // 16):
        rows[0, i, pl.ds(q * 16, 16)] = jnp.zeros((16,), jnp.float32)
      return carry

    lax.fori_loop(0, chunk, zrow, 0)
    for t in range(RPT // chunk if RPT % chunk == 0 else 0):
      pltpu.sync_copy(rows.at[0], acc.at[pl.ds(s * RPT + t * chunk, chunk)])
    if RPT % chunk:
      nz = RPT // 16

      def zcopy(t, carry):
        pltpu.sync_copy(rows.at[0, pl.ds(0, 16)],
                        acc.at[pl.ds(s * RPT + t * 16, 16)])
        return carry

      lax.fori_loop(0, nz, zcopy, 0)
    plsc.subcore_barrier()

    for b in range(NB):
      pltpu.async_copy(tab.at[sbuf.at[b]], rows.at[b], gsems[b])

    def outer(g, carry):
      jb = g * NB
      for b in range(NB):
        j = jb + b
        pltpu.make_async_copy(tab.at[sbuf.at[j]], rows.at[b],
                              gsems[b]).wait()
        pltpu.sync_copy(rows.at[b], acc.at[dbuf.at[j]], add=True)

        @pl.when(j + NB < nchunk)
        def _():
          pltpu.async_copy(tab.at[sbuf.at[j + NB]], rows.at[b], gsems[b])

      return carry

    lax.fori_loop(0, nchunk // NB, outer, 0)
    plsc.subcore_barrier()
    pltpu.sync_copy(acc.at[pl.ds(s * RPT, RPT)],
                    out_hbm.at[pl.ds(c * NPAD + s * RPT, RPT)])

  return k


_edge_pass_d = _make_edge_pass(D, CHUNK, NCHUNK)
_edge_pass_z = _make_edge_pass_spmem(Z, CHUNKW, NCHUNKW)
_deg_pass = _make_deg_pass()


def _dinv_from(dega, degb):
  deg = dega[0, :, 0] + degb[0, :, 0] + 1.0
  return lax.rsqrt(jnp.maximum(deg, 1e-12))


def _mlp_body(x, w1, b1, w2, b2, w0, dega, degb, out):
  dinv = _dinv_from(dega[...], degb[...])
  h = jax.nn.sigmoid(jnp.dot(x[...], w1[...],
                             preferred_element_type=jnp.float32) + b1[...])
  h = jax.nn.sigmoid(jnp.dot(h, w2[...],
                             preferred_element_type=jnp.float32) + b2[...])
  out[...] = dinv[:, None] * jnp.dot(h, w0[...],
                                     preferred_element_type=jnp.float32)


def _combine_body(spa, spb, ht, b, w, dega, degb, out):
  dinv = _dinv_from(dega[...], degb[...])
  o = dinv[:, None] * (spa[0] + spb[0] + ht[...]) + b[...]
  out[...] = dinv[:, None] * jnp.dot(o, w[...],
                                     preferred_element_type=jnp.float32)


def _final_body(spa, spb, ht, b, dega, degb, out):
  dinv = _dinv_from(dega[...], degb[...])
  out[...] = dinv[:, None] * (spa[0] + spb[0] + ht[...]) + b[...]


def _row_spec(feat):
  return pl.BlockSpec((ROW_BLK, feat), lambda i: (i, 0))


def _slab_specs(feat):
  return (pl.BlockSpec((1, ROW_BLK, feat), lambda i: (0, i, 0)),
          pl.BlockSpec((1, ROW_BLK, feat), lambda i: (1, i, 0)))


def _full_spec(shape):
  return pl.BlockSpec(shape, lambda i: tuple(0 for _ in shape))


def _tc_mlp(X, w1, b1, w2, b2, w0, degp):
  dega, degb = _slab_specs(Z)
  return pl.pallas_call(
      _mlp_body,
      grid=(GRID,),
      in_specs=[
          _row_spec(D), _full_spec((D, D)), _full_spec((D,)),
          _full_spec((D, D)), _full_spec((D,)), _full_spec((D, D)),
          dega, degb,
      ],
      out_specs=_row_spec(D),
      out_shape=jax.ShapeDtypeStruct((N, D), jnp.float32),
  )(X, w1, b1, w2, b2, w0, degp, degp)


def _tc_combine(sp, ht, b, w, w_out, degp):
  spa, spb = _slab_specs(D)
  dega, degb = _slab_specs(Z)
  return pl.pallas_call(
      _combine_body,
      grid=(GRID,),
      in_specs=[
          spa, spb, _row_spec(D), _full_spec((D,)),
          _full_spec((D, w_out)), dega, degb,
      ],
      out_specs=_row_spec(w_out),
      out_shape=jax.ShapeDtypeStruct((N, w_out), jnp.float32),
  )(sp, sp, ht, b, w, degp, degp)


def _tc_final(sp, ht, b, degp):
  spa, spb = _slab_specs(Z)
  dega, degb = _slab_specs(Z)
  return pl.pallas_call(
      _final_body,
      grid=(GRID,),
      in_specs=[spa, spb, _row_spec(Z), _full_spec((Z,)), dega, degb],
      out_specs=_row_spec(Z),
      out_shape=jax.ShapeDtypeStruct((N, Z), jnp.float32),
  )(sp, sp, ht, b, degp, degp)


def kernel(adj, X, fc1_W, fc1_b, fc2_W, fc2_b, gcn0_W, gcn0_b, gcn1_W,
           gcn1_b, assign_W, assign_b):
  src = adj[0].astype(jnp.int32)
  dst = adj[1].astype(jnp.int32)
  src_n = src.reshape(NW, NCHUNK, CHUNK)
  dst_n = dst.reshape(NW, NCHUNK, CHUNK)
  src_w = src.reshape(NW, NCHUNKW, CHUNKW)
  dst_w = dst.reshape(NW, NCHUNKW, CHUNKW)

  degp = _deg_pass(dst_w).reshape(NC, NPAD, Z)
  ht0 = _tc_mlp(X, fc1_W, fc1_b, fc2_W, fc2_b, gcn0_W, degp)
  sp0 = _edge_pass_d(src_n, dst_n, ht0).reshape(NC, NPAD, D)
  ht1 = _tc_combine(sp0, ht0, gcn0_b, gcn1_W, D, degp)
  sp1 = _edge_pass_d(src_n, dst_n, ht1).reshape(NC, NPAD, D)
  ht2 = _tc_combine(sp1, ht1, gcn1_b, assign_W, Z, degp)
  sp2 = _edge_pass_z(src_w, dst_w, ht2).reshape(NC, NPAD, Z)
  return _tc_final(sp2, ht2, assign_b, degp)


# speedup vs baseline: 1.6936x; 1.0597x over previous
import functools

import jax
import jax.numpy as jnp
from jax import lax
from jax.experimental import pallas as pl
from jax.experimental.pallas import tpu as pltpu
from jax.experimental.pallas import tpu_sc as plsc

N = 10000
E = 320000
D = 128
Z = 16

NC = 2
NS = 16
NW = NC * NS
EPW = E // NW
CHUNK = 40
NCHUNK = EPW // CHUNK
NB = 5
CHUNKW = 125
NCHUNKW = EPW // CHUNKW
NPAD = 10240
RPT = NPAD // NS
ZROWS = 128

ROW_BLK = 2000
GRID = N // ROW_BLK


def _make_edge_pass(feat, chunk, nchunk):
  mesh = plsc.VectorSubcoreMesh(core_axis_name="c", subcore_axis_name="s")

  @functools.partial(
      pl.kernel,
      mesh=mesh,
      out_type=jax.ShapeDtypeStruct((NC, NPAD, feat), jnp.float32),
      compiler_params=pltpu.CompilerParams(use_tc_tiling_on_sc=False),
      scratch_types=[
          pltpu.VMEM((nchunk, chunk), jnp.int32),
          pltpu.VMEM((nchunk, chunk), jnp.int32),
          pltpu.VMEM((NB, chunk, feat), jnp.float32),
          pltpu.VMEM_SHARED((NPAD, feat), jnp.float32),
      ] + [pltpu.SemaphoreType.DMA] * NB,
  )
  def k(adj_hbm, ht_hbm, out_hbm, sbuf, dbuf, rows, acc, *gsems):
    c = lax.axis_index("c")
    s = lax.axis_index("s")
    wid = s * NC + c

    pltpu.sync_copy(adj_hbm.at[0, wid], sbuf)
    pltpu.sync_copy(adj_hbm.at[1, wid], dbuf)

    def zrow(i, carry):
      for q in range(feat // 16):
        rows[0, i, pl.ds(q * 16, 16)] = jnp.zeros((16,), jnp.float32)
      return carry

    lax.fori_loop(0, chunk, zrow, 0)
    for t in range(RPT // chunk if RPT % chunk == 0 else 0):
      pltpu.sync_copy(rows.at[0], acc.at[pl.ds(s * RPT + t * chunk, chunk)])
    if RPT % chunk:
      nz = RPT // 16
      def zcopy(t, carry):
        pltpu.sync_copy(rows.at[0, pl.ds(0, 16)],
                        acc.at[pl.ds(s * RPT + t * 16, 16)])
        return carry
      lax.fori_loop(0, nz, zcopy, 0)
    plsc.subcore_barrier()

    for b in range(NB):
      pltpu.async_copy(ht_hbm.at[sbuf.at[b]], rows.at[b], gsems[b])

    def outer(g, carry):
      jb = g * NB
      for b in range(NB):
        j = jb + b
        pltpu.make_async_copy(ht_hbm.at[sbuf.at[j]], rows.at[b],
                              gsems[b]).wait()
        pltpu.sync_copy(rows.at[b], acc.at[dbuf.at[j]], add=True)

        @pl.when(j + NB < nchunk)
        def _():
          pltpu.async_copy(ht_hbm.at[sbuf.at[j + NB]], rows.at[b], gsems[b])

      return carry

    lax.fori_loop(0, nchunk // NB, outer, 0)
    plsc.subcore_barrier()
    pltpu.sync_copy(acc.at[pl.ds(s * RPT, RPT)],
                    out_hbm.at[c, pl.ds(s * RPT, RPT)])

  return k


def _make_deg_pass():
  mesh = plsc.VectorSubcoreMesh(core_axis_name="c", subcore_axis_name="s")

  @functools.partial(
      pl.kernel,
      mesh=mesh,
      out_type=jax.ShapeDtypeStruct((NC, NPAD, Z), jnp.float32),
      compiler_params=pltpu.CompilerParams(use_tc_tiling_on_sc=False),
      scratch_types=[
          pltpu.VMEM((NCHUNKW, CHUNKW), jnp.int32),
          pltpu.VMEM((CHUNKW, Z), jnp.float32),
          pltpu.VMEM((ZROWS, Z), jnp.float32),
          pltpu.VMEM_SHARED((NPAD, Z), jnp.float32),
          pltpu.SemaphoreType.DMA,
      ],
  )
  def k(adj_hbm, out_hbm, dbuf, ones, zbuf, acc, ssem):
    c = lax.axis_index("c")
    s = lax.axis_index("s")
    wid = s * NC + c

    pltpu.sync_copy(adj_hbm.at[1, wid], dbuf)

    def fill(i, carry):
      zbuf[i, pl.ds(0, 16)] = jnp.zeros((16,), jnp.float32)
      return carry

    lax.fori_loop(0, ZROWS, fill, 0)

    def fill1(i, carry):
      ones[i, pl.ds(0, 16)] = jnp.ones((16,), jnp.float32)
      return carry

    lax.fori_loop(0, CHUNKW, fill1, 0)
    for t in range(RPT // ZROWS):
      pltpu.sync_copy(zbuf, acc.at[pl.ds(s * RPT + t * ZROWS, ZROWS)])
    plsc.subcore_barrier()

    def outer(g, carry):
      jb = g * NB
      for b in range(NB):
        pltpu.async_copy(ones, acc.at[dbuf.at[jb + b]], ssem, add=True)
      for b in range(NB):
        pltpu.make_async_copy(ones, acc.at[dbuf.at[jb + b]], ssem).wait()
      return carry

    lax.fori_loop(0, NCHUNKW // NB, outer, 0)
    plsc.subcore_barrier()
    pltpu.sync_copy(acc.at[pl.ds(s * RPT, RPT)],
                    out_hbm.at[c, pl.ds(s * RPT, RPT)])

  return k


def _make_edge_pass_spmem(feat, chunk, nchunk):
  mesh = plsc.VectorSubcoreMesh(core_axis_name="c", subcore_axis_name="s")
  nrows = N // NS

  @functools.partial(
      pl.kernel,
      mesh=mesh,
      out_type=jax.ShapeDtypeStruct((NC, NPAD, feat), jnp.float32),
      compiler_params=pltpu.CompilerParams(use_tc_tiling_on_sc=False),
      scratch_types=[
          pltpu.VMEM((nchunk, chunk), jnp.int32),
          pltpu.VMEM((nchunk, chunk), jnp.int32),
          pltpu.VMEM((NB, chunk, feat), jnp.float32),
          pltpu.VMEM_SHARED((N, feat), jnp.float32),
          pltpu.VMEM_SHARED((NPAD, feat), jnp.float32),
      ] + [pltpu.SemaphoreType.DMA] * NB,
  )
  def k(adj_hbm, ht_hbm, out_hbm, sbuf, dbuf, rows, tab, acc,
        *gsems):
    c = lax.axis_index("c")
    s = lax.axis_index("s")
    wid = s * NC + c

    pltpu.sync_copy(adj_hbm.at[0, wid], sbuf)
    pltpu.sync_copy(adj_hbm.at[1, wid], dbuf)
    pltpu.sync_copy(ht_hbm.at[pl.ds(s * nrows, nrows)],
                    tab.at[pl.ds(s * nrows, nrows)])

    def zrow(i, carry):
      for q in range(feat // 16):
        rows[0, i, pl.ds(q * 16, 16)] = jnp.zeros((16,), jnp.float32)
      return carry

    lax.fori_loop(0, chunk, zrow, 0)
    for t in range(RPT // chunk if RPT % chunk == 0 else 0):
      pltpu.sync_copy(rows.at[0], acc.at[pl.ds(s * RPT + t * chunk, chunk)])
    if RPT % chunk:
      nz = RPT // 16

      def zcopy(t, carry):
        pltpu.sync_copy(rows.at[0, pl.ds(0, 16)],
                        acc.at[pl.ds(s * RPT + t * 16, 16)])
        return carry

      lax.fori_loop(0, nz, zcopy, 0)
    plsc.subcore_barrier()

    for b in range(NB):
      pltpu.async_copy(tab.at[sbuf.at[b]], rows.at[b], gsems[b])

    def outer(g, carry):
      jb = g * NB
      for b in range(NB):
        j = jb + b
        pltpu.make_async_copy(tab.at[sbuf.at[j]], rows.at[b],
                              gsems[b]).wait()
        pltpu.sync_copy(rows.at[b], acc.at[dbuf.at[j]], add=True)

        @pl.when(j + NB < nchunk)
        def _():
          pltpu.async_copy(tab.at[sbuf.at[j + NB]], rows.at[b], gsems[b])

      return carry

    lax.fori_loop(0, nchunk // NB, outer, 0)
    plsc.subcore_barrier()
    pltpu.sync_copy(acc.at[pl.ds(s * RPT, RPT)],
                    out_hbm.at[c, pl.ds(s * RPT, RPT)])

  return k


_edge_pass_d = _make_edge_pass(D, CHUNK, NCHUNK)
_edge_pass_z = _make_edge_pass_spmem(Z, CHUNKW, NCHUNKW)
_deg_pass = _make_deg_pass()


def _dinv_from(dega, degb):
  deg = dega[0, :, 0] + degb[0, :, 0] + 1.0
  return lax.rsqrt(jnp.maximum(deg, 1e-12))


def _mlp_body(x, w1, b1, w2, b2, w0, dega, degb, out):
  dinv = _dinv_from(dega[...], degb[...])
  h = jax.nn.sigmoid(jnp.dot(x[...], w1[...],
                             preferred_element_type=jnp.float32) + b1[...])
  h = jax.nn.sigmoid(jnp.dot(h, w2[...],
                             preferred_element_type=jnp.float32) + b2[...])
  out[...] = dinv[:, None] * jnp.dot(h, w0[...],
                                     preferred_element_type=jnp.float32)


def _combine_body(spa, spb, ht, b, w, dega, degb, out):
  dinv = _dinv_from(dega[...], degb[...])
  o = dinv[:, None] * (spa[0] + spb[0] + ht[...]) + b[...]
  out[...] = dinv[:, None] * jnp.dot(o, w[...],
                                     preferred_element_type=jnp.float32)


def _final_body(spa, spb, ht, b, dega, degb, out):
  dinv = _dinv_from(dega[...], degb[...])
  out[...] = dinv[:, None] * (spa[0] + spb[0] + ht[...]) + b[...]


def _row_spec(feat):
  return pl.BlockSpec((ROW_BLK, feat), lambda i: (i, 0))


def _slab_specs(feat):
  return (pl.BlockSpec((1, ROW_BLK, feat), lambda i: (0, i, 0)),
          pl.BlockSpec((1, ROW_BLK, feat), lambda i: (1, i, 0)))


def _full_spec(shape):
  return pl.BlockSpec(shape, lambda i: tuple(0 for _ in shape))


def _tc_mlp(X, w1, b1, w2, b2, w0, degp):
  dega, degb = _slab_specs(Z)
  return pl.pallas_call(
      _mlp_body,
      grid=(GRID,),
      in_specs=[
          _row_spec(D), _full_spec((D, D)), _full_spec((D,)),
          _full_spec((D, D)), _full_spec((D,)), _full_spec((D, D)),
          dega, degb,
      ],
      out_specs=_row_spec(D),
      out_shape=jax.ShapeDtypeStruct((N, D), jnp.float32),
  )(X, w1, b1, w2, b2, w0, degp, degp)


def _tc_combine(sp, ht, b, w, w_out, degp):
  spa, spb = _slab_specs(D)
  dega, degb = _slab_specs(Z)
  return pl.pallas_call(
      _combine_body,
      grid=(GRID,),
      in_specs=[
          spa, spb, _row_spec(D), _full_spec((D,)),
          _full_spec((D, w_out)), dega, degb,
      ],
      out_specs=_row_spec(w_out),
      out_shape=jax.ShapeDtypeStruct((N, w_out), jnp.float32),
  )(sp, sp, ht, b, w, degp, degp)


def _tc_final(sp, ht, b, degp):
  spa, spb = _slab_specs(Z)
  dega, degb = _slab_specs(Z)
  return pl.pallas_call(
      _final_body,
      grid=(GRID,),
      in_specs=[spa, spb, _row_spec(Z), _full_spec((Z,)), dega, degb],
      out_specs=_row_spec(Z),
      out_shape=jax.ShapeDtypeStruct((N, Z), jnp.float32),
  )(sp, sp, ht, b, degp, degp)


def kernel(adj, X, fc1_W, fc1_b, fc2_W, fc2_b, gcn0_W, gcn0_b, gcn1_W,
           gcn1_b, assign_W, assign_b):
  adj32 = adj.astype(jnp.int32)
  adj_n = adj32.reshape(2, NW, NCHUNK, CHUNK)
  adj_w = adj32.reshape(2, NW, NCHUNKW, CHUNKW)

  degp = _deg_pass(adj_w)
  ht0 = _tc_mlp(X, fc1_W, fc1_b, fc2_W, fc2_b, gcn0_W, degp)
  sp0 = _edge_pass_d(adj_n, ht0)
  ht1 = _tc_combine(sp0, ht0, gcn0_b, gcn1_W, D, degp)
  sp1 = _edge_pass_d(adj_n, ht1)
  ht2 = _tc_combine(sp1, ht1, gcn1_b, assign_W, Z, degp)
  sp2 = _edge_pass_z(adj_w, ht2)
  return _tc_final(sp2, ht2, assign_b, degp)


# speedup vs baseline: 1.7144x; 1.0123x over previous
import functools

import jax
import jax.numpy as jnp
from jax import lax
from jax.experimental import pallas as pl
from jax.experimental.pallas import tpu as pltpu
from jax.experimental.pallas import tpu_sc as plsc

N = 10000
E = 320000
D = 128
Z = 16

NC = 2
NS = 16
NW = NC * NS
EPW = E // NW
CHUNK = 40
NCHUNK = EPW // CHUNK
NB = 5
CHUNKW = 80
NCHUNKW = EPW // CHUNKW
NPAD = 10240
RPT = NPAD // NS
ZROWS = 128

ROW_BLK = 2000
GRID = N // ROW_BLK


def _make_edge_pass(feat, chunk, nchunk):
  mesh = plsc.VectorSubcoreMesh(core_axis_name="c", subcore_axis_name="s")

  @functools.partial(
      pl.kernel,
      mesh=mesh,
      out_type=jax.ShapeDtypeStruct((NC, NPAD, feat), jnp.float32),
      compiler_params=pltpu.CompilerParams(use_tc_tiling_on_sc=False),
      scratch_types=[
          pltpu.VMEM((EPW,), jnp.int32),
          pltpu.VMEM((EPW,), jnp.int32),
          pltpu.VMEM((NB, chunk, feat), jnp.float32),
          pltpu.VMEM_SHARED((NPAD, feat), jnp.float32),
      ] + [pltpu.SemaphoreType.DMA] * NB,
  )
  def k(adj_hbm, ht_hbm, out_hbm, sbuf, dbuf, rows, acc, *gsems):
    c = lax.axis_index("c")
    s = lax.axis_index("s")
    wid = s * NC + c

    pltpu.sync_copy(adj_hbm.at[0, pl.ds(wid * EPW, EPW)], sbuf)
    pltpu.sync_copy(adj_hbm.at[1, pl.ds(wid * EPW, EPW)], dbuf)

    def zrow(i, carry):
      for q in range(feat // 16):
        rows[0, i, pl.ds(q * 16, 16)] = jnp.zeros((16,), jnp.float32)
      return carry

    lax.fori_loop(0, chunk, zrow, 0)
    for t in range(RPT // chunk if RPT % chunk == 0 else 0):
      pltpu.sync_copy(rows.at[0], acc.at[pl.ds(s * RPT + t * chunk, chunk)])
    if RPT % chunk:
      nz = RPT // 16
      def zcopy(t, carry):
        pltpu.sync_copy(rows.at[0, pl.ds(0, 16)],
                        acc.at[pl.ds(s * RPT + t * 16, 16)])
        return carry
      lax.fori_loop(0, nz, zcopy, 0)
    plsc.subcore_barrier()

    def sidx(j):
      return sbuf.at[pl.ds(j * chunk, chunk)]

    def didx(j):
      return dbuf.at[pl.ds(j * chunk, chunk)]

    for b in range(NB):
      pltpu.async_copy(ht_hbm.at[sidx(b)], rows.at[b], gsems[b])

    def outer(g, carry):
      jb = g * NB
      for b in range(NB):
        j = jb + b
        pltpu.make_async_copy(ht_hbm.at[sidx(j)], rows.at[b],
                              gsems[b]).wait()
        pltpu.sync_copy(rows.at[b], acc.at[didx(j)], add=True)

        @pl.when(j + NB < nchunk)
        def _():
          pltpu.async_copy(ht_hbm.at[sidx(j + NB)], rows.at[b], gsems[b])

      return carry

    lax.fori_loop(0, nchunk // NB, outer, 0)
    plsc.subcore_barrier()
    pltpu.sync_copy(acc.at[pl.ds(s * RPT, RPT)],
                    out_hbm.at[c, pl.ds(s * RPT, RPT)])

  return k


def _make_deg_pass():
  mesh = plsc.VectorSubcoreMesh(core_axis_name="c", subcore_axis_name="s")

  @functools.partial(
      pl.kernel,
      mesh=mesh,
      out_type=jax.ShapeDtypeStruct((NC, NPAD, Z), jnp.float32),
      compiler_params=pltpu.CompilerParams(use_tc_tiling_on_sc=False),
      scratch_types=[
          pltpu.VMEM((EPW,), jnp.int32),
          pltpu.VMEM((CHUNKW, Z), jnp.float32),
          pltpu.VMEM((ZROWS, Z), jnp.float32),
          pltpu.VMEM_SHARED((NPAD, Z), jnp.float32),
          pltpu.SemaphoreType.DMA,
      ],
  )
  def k(adj_hbm, out_hbm, dbuf, ones, zbuf, acc, ssem):
    c = lax.axis_index("c")
    s = lax.axis_index("s")
    wid = s * NC + c

    pltpu.sync_copy(adj_hbm.at[1, pl.ds(wid * EPW, EPW)], dbuf)

    def fill(i, carry):
      zbuf[i, pl.ds(0, 16)] = jnp.zeros((16,), jnp.float32)
      return carry

    lax.fori_loop(0, ZROWS, fill, 0)

    def fill1(i, carry):
      ones[i, pl.ds(0, 16)] = jnp.ones((16,), jnp.float32)
      return carry

    lax.fori_loop(0, CHUNKW, fill1, 0)
    for t in range(RPT // ZROWS):
      pltpu.sync_copy(zbuf, acc.at[pl.ds(s * RPT + t * ZROWS, ZROWS)])
    plsc.subcore_barrier()

    def outer(g, carry):
      jb = g * NB
      for b in range(NB):
        pltpu.async_copy(ones, acc.at[dbuf.at[pl.ds((jb + b) * CHUNKW,
                                                    CHUNKW)]],
                         ssem, add=True)
      for b in range(NB):
        pltpu.make_async_copy(ones, acc.at[dbuf.at[pl.ds((jb + b) * CHUNKW,
                                                         CHUNKW)]],
                              ssem).wait()
      return carry

    lax.fori_loop(0, NCHUNKW // NB, outer, 0)
    plsc.subcore_barrier()
    pltpu.sync_copy(acc.at[pl.ds(s * RPT, RPT)],
                    out_hbm.at[c, pl.ds(s * RPT, RPT)])

  return k


def _make_edge_pass_spmem(feat, chunk, nchunk):
  mesh = plsc.VectorSubcoreMesh(core_axis_name="c", subcore_axis_name="s")
  nrows = N // NS

  @functools.partial(
      pl.kernel,
      mesh=mesh,
      out_type=jax.ShapeDtypeStruct((NC, NPAD, feat), jnp.float32),
      compiler_params=pltpu.CompilerParams(use_tc_tiling_on_sc=False),
      scratch_types=[
          pltpu.VMEM((EPW,), jnp.int32),
          pltpu.VMEM((EPW,), jnp.int32),
          pltpu.VMEM((NB, chunk, feat), jnp.float32),
          pltpu.VMEM_SHARED((N, feat), jnp.float32),
          pltpu.VMEM_SHARED((NPAD, feat), jnp.float32),
      ] + [pltpu.SemaphoreType.DMA] * NB,
  )
  def k(adj_hbm, ht_hbm, out_hbm, sbuf, dbuf, rows, tab, acc,
        *gsems):
    c = lax.axis_index("c")
    s = lax.axis_index("s")
    wid = s * NC + c

    pltpu.sync_copy(adj_hbm.at[0, pl.ds(wid * EPW, EPW)], sbuf)
    pltpu.sync_copy(adj_hbm.at[1, pl.ds(wid * EPW, EPW)], dbuf)
    pltpu.sync_copy(ht_hbm.at[pl.ds(s * nrows, nrows)],
                    tab.at[pl.ds(s * nrows, nrows)])

    def zrow(i, carry):
      for q in range(feat // 16):
        rows[0, i, pl.ds(q * 16, 16)] = jnp.zeros((16,), jnp.float32)
      return carry

    lax.fori_loop(0, chunk, zrow, 0)
    for t in range(RPT // chunk if RPT % chunk == 0 else 0):
      pltpu.sync_copy(rows.at[0], acc.at[pl.ds(s * RPT + t * chunk, chunk)])
    if RPT % chunk:
      nz = RPT // 16

      def zcopy(t, carry):
        pltpu.sync_copy(rows.at[0, pl.ds(0, 16)],
                        acc.at[pl.ds(s * RPT + t * 16, 16)])
        return carry

      lax.fori_loop(0, nz, zcopy, 0)
    plsc.subcore_barrier()

    def sidx(j):
      return sbuf.at[pl.ds(j * chunk, chunk)]

    def didx(j):
      return dbuf.at[pl.ds(j * chunk, chunk)]

    for b in range(NB):
      pltpu.async_copy(tab.at[sidx(b)], rows.at[b], gsems[b])

    def outer(g, carry):
      jb = g * NB
      for b in range(NB):
        j = jb + b
        pltpu.make_async_copy(tab.at[sidx(j)], rows.at[b],
                              gsems[b]).wait()
        pltpu.sync_copy(rows.at[b], acc.at[didx(j)], add=True)

        @pl.when(j + NB < nchunk)
        def _():
          pltpu.async_copy(tab.at[sidx(j + NB)], rows.at[b], gsems[b])

      return carry

    lax.fori_loop(0, nchunk // NB, outer, 0)
    plsc.subcore_barrier()
    pltpu.sync_copy(acc.at[pl.ds(s * RPT, RPT)],
                    out_hbm.at[c, pl.ds(s * RPT, RPT)])

  return k


_edge_pass_d = _make_edge_pass(D, CHUNK, NCHUNK)
_edge_pass_z = _make_edge_pass_spmem(Z, CHUNKW, NCHUNKW)
_deg_pass = _make_deg_pass()


def _dinv_from(dega, degb):
  deg = dega[0, :, 0] + degb[0, :, 0] + 1.0
  return lax.rsqrt(jnp.maximum(deg, 1e-12))


def _mlp_body(x, w1, b1, w2, b2, w0, dega, degb, out):
  dinv = _dinv_from(dega[...], degb[...])
  h = jax.nn.sigmoid(jnp.dot(x[...], w1[...],
                             preferred_element_type=jnp.float32) + b1[...])
  h = jax.nn.sigmoid(jnp.dot(h, w2[...],
                             preferred_element_type=jnp.float32) + b2[...])
  out[...] = dinv[:, None] * jnp.dot(h, w0[...],
                                     preferred_element_type=jnp.float32)


def _combine_body(spa, spb, ht, b, w, dega, degb, out):
  dinv = _dinv_from(dega[...], degb[...])
  o = dinv[:, None] * (spa[0] + spb[0] + ht[...]) + b[...]
  out[...] = dinv[:, None] * jnp.dot(o, w[...],
                                     preferred_element_type=jnp.float32)


def _final_body(spa, spb, ht, b, dega, degb, out):
  dinv = _dinv_from(dega[...], degb[...])
  out[...] = dinv[:, None] * (spa[0] + spb[0] + ht[...]) + b[...]


def _row_spec(feat):
  return pl.BlockSpec((ROW_BLK, feat), lambda i: (i, 0))


def _slab_specs(feat):
  return (pl.BlockSpec((1, ROW_BLK, feat), lambda i: (0, i, 0)),
          pl.BlockSpec((1, ROW_BLK, feat), lambda i: (1, i, 0)))


def _full_spec(shape):
  return pl.BlockSpec(shape, lambda i: tuple(0 for _ in shape))


def _tc_mlp(X, w1, b1, w2, b2, w0, degp):
  dega, degb = _slab_specs(Z)
  return pl.pallas_call(
      _mlp_body,
      grid=(GRID,),
      in_specs=[
          _row_spec(D), _full_spec((D, D)), _full_spec((D,)),
          _full_spec((D, D)), _full_spec((D,)), _full_spec((D, D)),
          dega, degb,
      ],
      out_specs=_row_spec(D),
      out_shape=jax.ShapeDtypeStruct((N, D), jnp.float32),
  )(X, w1, b1, w2, b2, w0, degp, degp)


def _tc_combine(sp, ht, b, w, w_out, degp):
  spa, spb = _slab_specs(D)
  dega, degb = _slab_specs(Z)
  return pl.pallas_call(
      _combine_body,
      grid=(GRID,),
      in_specs=[
          spa, spb, _row_spec(D), _full_spec((D,)),
          _full_spec((D, w_out)), dega, degb,
      ],
      out_specs=_row_spec(w_out),
      out_shape=jax.ShapeDtypeStruct((N, w_out), jnp.float32),
  )(sp, sp, ht, b, w, degp, degp)


def _tc_final(sp, ht, b, degp):
  spa, spb = _slab_specs(Z)
  dega, degb = _slab_specs(Z)
  return pl.pallas_call(
      _final_body,
      grid=(GRID,),
      in_specs=[spa, spb, _row_spec(Z), _full_spec((Z,)), dega, degb],
      out_specs=_row_spec(Z),
      out_shape=jax.ShapeDtypeStruct((N, Z), jnp.float32),
  )(sp, sp, ht, b, degp, degp)


def kernel(adj, X, fc1_W, fc1_b, fc2_W, fc2_b, gcn0_W, gcn0_b, gcn1_W,
           gcn1_b, assign_W, assign_b):
  adj32 = adj.astype(jnp.int32)

  degp = _deg_pass(adj32)
  ht0 = _tc_mlp(X, fc1_W, fc1_b, fc2_W, fc2_b, gcn0_W, degp)
  sp0 = _edge_pass_d(adj32, ht0)
  ht1 = _tc_combine(sp0, ht0, gcn0_b, gcn1_W, D, degp)
  sp1 = _edge_pass_d(adj32, ht1)
  ht2 = _tc_combine(sp1, ht1, gcn1_b, assign_W, Z, degp)
  sp2 = _edge_pass_z(adj32, ht2)
  return _tc_final(sp2, ht2, assign_b, degp)
